# Initial kernel scaffold; baseline (speedup 1.0000x reference)
import jax
import jax.numpy as jnp
from jax.experimental import pallas as pl


def kernel(x, edge_index, batch, params):
    raise NotImplementedError("write your pallas kernel here")


# speedup vs baseline: 10.6785x; 10.6785x over previous
import functools

import jax
import jax.numpy as jnp
from jax import lax
from jax.experimental import pallas as pl
from jax.experimental.pallas import tpu as pltpu
from jax.experimental.pallas import tpu_sc as plsc

N = 10000
D = 128
G = 64
NC = 2
NS = 16
NW = NC * NS
K = 128
NPAD = 10240
ROWS_PER_TILE = NPAD // NS

_F32 = jnp.float32
_HIGH = lax.Precision.HIGHEST

@functools.cache
def _mesh():
    return plsc.VectorSubcoreMesh(
        core_axis_name="c", subcore_axis_name="s", num_cores=NC, num_subcores=NS
    )


def _dot(a, b):
    return jnp.dot(a, b, preferred_element_type=_F32, precision=_HIGH)


def _ln(x, s, b, eps=1e-5):
    m = jnp.mean(x, axis=-1, keepdims=True)
    v = jnp.mean((x - m) ** 2, axis=-1, keepdims=True)
    return (x - m) * lax.rsqrt(v + eps) * s + b


def _dinv_of(hist):
    deg = 1.0 + hist[0, :N, 0:1] + hist[1, :N, 0:1]
    return lax.rsqrt(deg)


def _hist_kernel(epad):
    ept = epad // NW
    n_z = ROWS_PER_TILE // 64

    @functools.partial(
        pl.kernel,
        out_type=jax.ShapeDtypeStruct((NC, NPAD, 16), _F32),
        mesh=_mesh(),
        scratch_types=[
            pltpu.VMEM((1, K), jnp.int32),
            pltpu.VMEM((K, 16), _F32),
            pltpu.VMEM((64, 16), _F32),
            pltpu.VMEM_SHARED((NPAD, 16), _F32),
        ],
    )
    def hist(dst_hbm, out_hbm, dstbuf, ones_v, zbuf, acc):
        cid = lax.axis_index("c")
        sid = lax.axis_index("s")
        one = jnp.ones((16,), _F32)
        zero = jnp.zeros((16,), _F32)
        for i in range(K):
            ones_v[i, pl.ds(0, 16)] = one
        for i in range(64):
            zbuf[i, pl.ds(0, 16)] = zero
        rowbase = sid * ROWS_PER_TILE

        @pl.loop(0, n_z)
        def _(r):
            pltpu.sync_copy(zbuf, acc.at[pl.ds(rowbase + r * 64, 64)])

        plsc.subcore_barrier()
        base = (cid * NS + sid) * ept

        @pl.loop(0, ept, step=K)
        def _(e):
            pltpu.sync_copy(dst_hbm.at[pl.ds(base + e, K)], dstbuf.at[0])
            pltpu.sync_copy(ones_v, acc.at[dstbuf.at[0]], add=True)

        plsc.subcore_barrier()
        pltpu.sync_copy(
            acc.at[pl.ds(rowbase, ROWS_PER_TILE)],
            out_hbm.at[cid, pl.ds(rowbase, ROWS_PER_TILE)],
        )

    return hist


def _agg_kernel(epad):
    ept = epad // NW
    n_z = ROWS_PER_TILE // 16

    @functools.partial(
        pl.kernel,
        out_type=jax.ShapeDtypeStruct((NC, NPAD, D), _F32),
        mesh=_mesh(),
        scratch_types=[
            pltpu.VMEM((1, K), jnp.int32),
            pltpu.VMEM((1, K), jnp.int32),
            pltpu.VMEM((K, D), _F32),
            pltpu.VMEM((16, D), _F32),
            pltpu.VMEM_SHARED((NPAD, D), _F32),
            pltpu.SemaphoreType.DMA,
        ],
    )
    def agg(h_hbm, src_hbm, dst_hbm, out_hbm, srcbuf, dstbuf, rows, zbuf, acc,
            sem):
        cid = lax.axis_index("c")
        sid = lax.axis_index("s")
        zero = jnp.zeros((16,), _F32)
        for i in range(16):
            for j in range(8):
                zbuf[i, pl.ds(j * 16, 16)] = zero
        rowbase = sid * ROWS_PER_TILE

        @pl.loop(0, n_z)
        def _(r):
            pltpu.sync_copy(zbuf, acc.at[pl.ds(rowbase + r * 16, 16)])

        plsc.subcore_barrier()
        base = (cid * NS + sid) * ept

        @pl.loop(0, ept, step=K)
        def _(e):
            pltpu.sync_copy(src_hbm.at[pl.ds(base + e, K)], srcbuf.at[0])
            pltpu.sync_copy(dst_hbm.at[pl.ds(base + e, K)], dstbuf.at[0])
            pltpu.async_copy(h_hbm.at[srcbuf.at[0]], rows, sem).wait()
            pltpu.sync_copy(rows, acc.at[dstbuf.at[0]], add=True)

        plsc.subcore_barrier()
        pltpu.sync_copy(
            acc.at[pl.ds(rowbase, ROWS_PER_TILE)],
            out_hbm.at[cid, pl.ds(rowbase, ROWS_PER_TILE)],
        )

    return agg


def _tc_mm_body(x_ref, w_ref, o_ref):
    o_ref[...] = _dot(x_ref[...], w_ref[...])


def _tc_scale_body(h_ref, hist_ref, o_ref):
    o_ref[...] = h_ref[...] * _dinv_of(hist_ref[...])


def _tc_mid_body(p_ref, hp_ref, hist_ref, w_ref, b_ref, s_ref, sb_ref, o_ref):
    dinv = _dinv_of(hist_ref[...])
    tmp = p_ref[0, :N, :] + p_ref[1, :N, :] + hp_ref[...]
    a = jnp.tanh(tmp * dinv + b_ref[...])
    l = _ln(a, s_ref[...], sb_ref[...])
    o_ref[...] = _dot(l, w_ref[...]) * dinv


def _tc_node_body(p_ref, hp_ref, hist_ref, b2, s2, sb2, pns, pnb,
                  gw1, gb1, gw2, gb2, gw3, gb3, gw4, gb4,
                  hp_out, g_out):
    dinv = _dinv_of(hist_ref[...])
    tmp = p_ref[0, :N, :] + p_ref[1, :N, :] + hp_ref[...]
    a = jnp.tanh(tmp * dinv + b2[...])
    h = _ln(a, s2[...], sb2[...])
    hp = _ln(h, pns[...], pnb[...])
    g = jnp.tanh(_dot(hp, gw1[...]) + gb1[...])
    g = jnp.tanh(_dot(g, gw2[...]) + gb2[...])
    g = jnp.tanh(_dot(g, gw3[...]) + gb3[...])
    g = _dot(g, gw4[...]) + gb4[...]
    hp_out[...] = hp
    g_out[...] = g


def _tc_pool_body(hp_ref, g_ref, batch_ref,
                  mw1, mb1, m1s, m1b, mw2, mb2, m2s, m2b, mw3, mb3, o_ref):
    hp = hp_ref[...]
    g = g_ref[...]
    onehot = (batch_ref[...] ==
              lax.broadcasted_iota(jnp.int32, (N, G), 1)).astype(_F32)
    gm = jnp.max(jnp.where(onehot > 0.0, g, -jnp.inf), axis=0)
    gm = jnp.where(jnp.isfinite(gm), gm, 0.0)
    gmb = _dot(onehot, gm[:, None])
    e = jnp.exp(g - gmb)
    esum = lax.dot_general(onehot, e, (((0,), (0,)), ((), ())),
                           precision=_HIGH, preferred_element_type=_F32)
    esb = _dot(onehot, esum)
    alpha = e / (esb + 1e-16)
    pooled = lax.dot_general(onehot, alpha * hp, (((0,), (0,)), ((), ())),
                             precision=_HIGH, preferred_element_type=_F32)
    m = _dot(pooled, mw1[...]) + mb1[...]
    m = jnp.tanh(_ln(m, m1s[...], m1b[...]))
    m = _dot(m, mw2[...]) + mb2[...]
    m = jnp.tanh(_ln(m, m2s[...], m2b[...]))
    o_ref[...] = _dot(m, mw3[...]) + mb3[...]


def _tc_call(body, out_shape, *args):
    if isinstance(out_shape, tuple):
        out_shape = jax.ShapeDtypeStruct(out_shape, _F32)
    return pl.pallas_call(body, out_shape=out_shape)(*args)


def kernel(x, edge_index, batch, params):
    p = params
    e_num = edge_index.shape[1]
    epad = ((e_num + NW * K - 1) // (NW * K)) * (NW * K)
    pad = epad - e_num
    src = edge_index[0].astype(jnp.int32)
    dst = edge_index[1].astype(jnp.int32)
    if pad:
        src = jnp.concatenate([src, jnp.zeros((pad,), jnp.int32)])
        dst = jnp.concatenate([dst, jnp.full((pad,), N, jnp.int32)])
    batch2d = batch.astype(jnp.int32).reshape(N, 1)

    def r2(v):
        return v.reshape(1, -1)

    hist_fn = _hist_kernel(epad)
    agg_fn = _agg_kernel(epad)

    hist = hist_fn(dst)

    h1 = _tc_call(_tc_mm_body, (N, D), x, p['W1'])
    h1p = _tc_call(_tc_scale_body, (N, D), h1, hist)
    p1 = agg_fn(h1p, src, dst)
    h2p = _tc_call(_tc_mid_body, (N, D), p1, h1p, hist, p['W2'],
                   r2(p['b1']), r2(p['ln1_s']), r2(p['ln1_b']))
    p2 = agg_fn(h2p, src, dst)
    hp, g = _tc_call(
        _tc_node_body,
        [jax.ShapeDtypeStruct((N, D), _F32),
         jax.ShapeDtypeStruct((N, 1), _F32)],
        p2, h2p, hist,
      r2(p['b2']), r2(p['ln2_s']), r2(p['ln2_b']),
      r2(p['pn_s']), r2(p['pn_b']),
      p['Gw1'], r2(p['Gb1']), p['Gw2'], r2(p['Gb2']),
      p['Gw3'], r2(p['Gb3']), p['Gw4'], r2(p['Gb4']),
    )
    out = _tc_call(
        _tc_pool_body, (G, 64), hp, g, batch2d,
        p['Mw1'], r2(p['Mb1']), r2(p['Mln1_s']), r2(p['Mln1_b']),
        p['Mw2'], r2(p['Mb2']), r2(p['Mln2_s']), r2(p['Mln2_b']),
        p['Mw3'], r2(p['Mb3']),
    )
    return out
